# baseline (speedup 1.0000x reference)
import functools

import jax
import jax.numpy as jnp
from jax import lax
from jax.experimental import pallas as pl
from jax.experimental.pallas import tpu as pltpu
from jax.experimental.pallas import tpu_sc as plsc

_D = 256
_B = 16 * 1024
_CHUNK = 128

_info = plsc.get_sparse_core_info()
_NC, _NS = _info.num_cores, _info.num_subcores
_NW = _NC * _NS
_BPW = _B // _NW
_NCHUNK = _BPW // _CHUNK

_mesh = plsc.VectorSubcoreMesh(core_axis_name="c", subcore_axis_name="s")


_NBUF = 3


_ROWS, _COLS = 16, 1024
_WPR = _COLS // _BPW


@functools.partial(
    pl.kernel,
    mesh=_mesh,
    out_type=jax.ShapeDtypeStruct((_ROWS, _COLS, _D), jnp.float32),
    scratch_types=[
        pltpu.VMEM((_BPW,), jnp.int32),
        pltpu.VMEM((_NBUF * _CHUNK, _D), jnp.float32),
        pltpu.SemaphoreType.DMA,
        pltpu.SemaphoreType.DMA,
    ],
)
def _gather_sc(table_hbm, idx_hbm, out_hbm, idx_v, rows_v, gsem, osem):
    wid = lax.axis_index("s") * _NC + lax.axis_index("c")
    r = wid // _WPR
    c0 = (wid % _WPR) * _BPW
    half = _BPW // 2
    pltpu.sync_copy(idx_hbm.at[r, pl.ds(c0, half)], idx_v.at[pl.ds(0, half)])

    def gather(j, buf):
        return pltpu.async_copy(
            table_hbm.at[idx_v.at[pl.ds(j * _CHUNK, _CHUNK)]],
            rows_v.at[pl.ds(buf * _CHUNK, _CHUNK)], gsem)

    g0 = gather(0, 0)
    g1 = gather(1, 1)
    pltpu.sync_copy(idx_hbm.at[r, pl.ds(c0 + half, half)],
                    idx_v.at[pl.ds(half, half)])
    g2 = gather(2, 2)
    def scatter(j, buf, n=_CHUNK):
        return pltpu.async_copy(
            rows_v.at[pl.ds(buf * _CHUNK, n)],
            out_hbm.at[r, pl.ds(c0 + j * _CHUNK, n)], osem)

    g0.wait()
    o0 = scatter(0, 0)
    g1.wait()
    o1 = scatter(1, 1)
    o0.wait()
    g3 = gather(3, 0)
    g2.wait()
    o2 = scatter(2, 2)
    g3.wait()
    o3 = scatter(3, 0)
    o1.wait()
    o2.wait()
    o3.wait()


_V = 8192
_TCOLS = 4096


def _tr_body(w_ref, t_ref):
    t_ref[...] = w_ref[...].T


_transpose_tc = pl.pallas_call(
    _tr_body,
    grid=(_V // _TCOLS,),
    in_specs=[pl.BlockSpec((_D, _TCOLS), lambda i: (0, i))],
    out_specs=pl.BlockSpec((_TCOLS, _D), lambda i: (i, 0)),
    out_shape=jax.ShapeDtypeStruct((_V, _D), jnp.float32),
)


def kernel(embed_id, weight):
    table = _transpose_tc(weight)
    return _gather_sc(table, embed_id)


# speedup vs baseline: 1.0123x; 1.0123x over previous
import functools

import jax
import jax.numpy as jnp
from jax import lax
from jax.experimental import pallas as pl
from jax.experimental.pallas import tpu as pltpu
from jax.experimental.pallas import tpu_sc as plsc

_D = 256
_B = 16 * 1024
_CHUNK = 128

_info = plsc.get_sparse_core_info()
_NC, _NS = _info.num_cores, _info.num_subcores
_NW = _NC * _NS
_BPW = _B // _NW
_NCHUNK = _BPW // _CHUNK

_mesh = plsc.VectorSubcoreMesh(core_axis_name="c", subcore_axis_name="s")


_NBUF = 3


_ROWS, _COLS = 16, 1024
_WPR = _COLS // _BPW


@functools.partial(
    pl.kernel,
    mesh=_mesh,
    out_type=jax.ShapeDtypeStruct((_ROWS, _COLS, _D), jnp.float32),
    scratch_types=[
        pltpu.VMEM((_BPW,), jnp.int32),
        pltpu.VMEM((_NBUF * _CHUNK, _D), jnp.float32),
        pltpu.SemaphoreType.DMA,
        pltpu.SemaphoreType.DMA,
    ],
)
def _gather_sc(table_hbm, idx_hbm, out_hbm, idx_v, rows_v, gsem, osem):
    wid = lax.axis_index("s") * _NC + lax.axis_index("c")
    r = wid // _WPR
    c0 = (wid % _WPR) * _BPW
    half = _BPW // 2
    pltpu.sync_copy(idx_hbm.at[r, pl.ds(c0, half)], idx_v.at[pl.ds(0, half)])

    def gather(j, buf):
        return pltpu.async_copy(
            table_hbm.at[idx_v.at[pl.ds(j * _CHUNK, _CHUNK)]],
            rows_v.at[pl.ds(buf * _CHUNK, _CHUNK)], gsem)

    g0 = gather(0, 0)
    g1 = gather(1, 1)
    pltpu.sync_copy(idx_hbm.at[r, pl.ds(c0 + half, half)],
                    idx_v.at[pl.ds(half, half)])
    g2 = gather(2, 2)
    g0.wait()
    g1.wait()
    o01 = pltpu.async_copy(
        rows_v.at[pl.ds(0, 2 * _CHUNK)],
        out_hbm.at[r, pl.ds(c0, 2 * _CHUNK)], osem)
    o01.wait()
    g3 = gather(3, 0)
    g2.wait()
    o2 = pltpu.async_copy(
        rows_v.at[pl.ds(2 * _CHUNK, _CHUNK)],
        out_hbm.at[r, pl.ds(c0 + 2 * _CHUNK, _CHUNK)], osem)
    g3.wait()
    o3 = pltpu.async_copy(
        rows_v.at[pl.ds(0, _CHUNK)],
        out_hbm.at[r, pl.ds(c0 + 3 * _CHUNK, _CHUNK)], osem)
    o2.wait()
    o3.wait()


_V = 8192
_TCOLS = 4096


def _tr_body(w_ref, t_ref):
    t_ref[...] = w_ref[...].T


_transpose_tc = pl.pallas_call(
    _tr_body,
    grid=(_V // _TCOLS,),
    in_specs=[pl.BlockSpec((_D, _TCOLS), lambda i: (0, i))],
    out_specs=pl.BlockSpec((_TCOLS, _D), lambda i: (i, 0)),
    out_shape=jax.ShapeDtypeStruct((_V, _D), jnp.float32),
)


def kernel(embed_id, weight):
    table = _transpose_tc(weight)
    return _gather_sc(table, embed_id)
